# baseline (speedup 1.0000x reference)
import jax
import jax.numpy as jnp
from jax.experimental import pallas as pl
from jax.experimental.pallas import tpu as pltpu

_LQ = 200
_BBL = 256
_NC = _LQ // 8


def _rosa_block(u_ref, x_ref, v_ref, o_ref):
    xq = x_ref[...]
    vq = v_ref[...]
    u = u_ref[0, 0]

    rows8 = jax.lax.broadcasted_iota(jnp.int32, (8, 1), 0)
    xch = [xq[8 * c:8 * c + 8, :] for c in range(_NC)]
    chunks = [jnp.full((8, _BBL), u, dtype=jnp.float32)] * _NC

    for tp in range(_LQ - 1):
        xc = xq[tp:tp + 1, :]
        vc = vq[tp:tp + 1, :]
        ci = (tp + 1) // 8
        for c in range(ci, _NC):
            m = xch[c] == xc
            if c * 8 <= tp:
                m = m & (rows8 + c * 8 > tp)
            chunks[c] = jnp.where(m, vc, chunks[c])

    o_ref[...] = jnp.concatenate(chunks, axis=0).T


def kernel(x, v, u):
    B, L = x.shape
    xT = x.astype(jnp.int32).T
    vT = v.T
    u_arr = jnp.full((1, 1), u, dtype=jnp.float32)

    out = pl.pallas_call(
        _rosa_block,
        grid=(B // _BBL,),
        in_specs=[
            pl.BlockSpec(memory_space=pltpu.SMEM),
            pl.BlockSpec((L, _BBL), lambda i: (0, i)),
            pl.BlockSpec((L, _BBL), lambda i: (0, i)),
        ],
        out_specs=pl.BlockSpec((_BBL, L), lambda i: (i, 0)),
        out_shape=jax.ShapeDtypeStruct((B, L), jnp.float32),
        compiler_params=pltpu.CompilerParams(
            dimension_semantics=("parallel",)),
    )(u_arr, xT, vT)
    return out


# speedup vs baseline: 1.1114x; 1.1114x over previous
import jax
import jax.numpy as jnp
from jax.experimental import pallas as pl
from jax.experimental.pallas import tpu as pltpu

_LQ = 200
_BBL = 256
_NC = _LQ // 8


def _rosa_block(u_ref, xv_ref, o_ref):
    xq = xv_ref[0]
    vq = jax.lax.bitcast_convert_type(xv_ref[1], jnp.float32)
    u = u_ref[0, 0]

    rows8 = jax.lax.broadcasted_iota(jnp.int32, (8, 1), 0)
    xch = [xq[8 * c:8 * c + 8, :] for c in range(_NC)]
    chunks = [jnp.full((8, _BBL), u, dtype=jnp.float32)] * _NC

    for tp in range(_LQ - 1):
        xc = xq[tp:tp + 1, :]
        vc = vq[tp:tp + 1, :]
        ci = (tp + 1) // 8
        for c in range(ci, _NC):
            m = xch[c] == xc
            if c * 8 <= tp:
                m = m & (rows8 + c * 8 > tp)
            chunks[c] = jnp.where(m, vc, chunks[c])

    o_ref[...] = jnp.concatenate(chunks, axis=0)


def kernel(x, v, u):
    B, L = x.shape
    xv = jnp.stack([x.astype(jnp.int32),
                    jax.lax.bitcast_convert_type(v, jnp.int32)])
    xvT = jnp.transpose(xv, (0, 2, 1))
    u_arr = jnp.full((1, 1), u, dtype=jnp.float32)

    out = pl.pallas_call(
        _rosa_block,
        grid=(B // _BBL,),
        in_specs=[
            pl.BlockSpec(memory_space=pltpu.SMEM),
            pl.BlockSpec((2, L, _BBL), lambda i: (0, 0, i)),
        ],
        out_specs=pl.BlockSpec((L, _BBL), lambda i: (0, i)),
        out_shape=jax.ShapeDtypeStruct((L, B), jnp.float32),
        compiler_params=pltpu.CompilerParams(
            dimension_semantics=("parallel",)),
    )(u_arr, xvT)
    return out.T


# speedup vs baseline: 1.4155x; 1.2737x over previous
import jax
import jax.numpy as jnp
from jax.experimental import pallas as pl
from jax.experimental.pallas import tpu as pltpu

_LQ = 200
_BBL = 256
_NC = _LQ // 8


def _rosa_block(u_ref, x_ref, v_ref, o_ref):
    xq = x_ref[...]
    vq = v_ref[...]
    u = u_ref[0, 0]

    rows8 = jax.lax.broadcasted_iota(jnp.int32, (8, 1), 0)
    xch = [xq[8 * c:8 * c + 8, :] for c in range(_NC)]
    chunks = [jnp.full((8, _BBL), u, dtype=jnp.float32)] * _NC

    for tp in range(_LQ - 1):
        xc = xq[tp:tp + 1, :]
        vc = vq[tp:tp + 1, :]
        ci = (tp + 1) // 8
        for c in range(ci, _NC):
            m = xch[c] == xc
            if c * 8 <= tp:
                m = m & (rows8 + c * 8 > tp)
            chunks[c] = jnp.where(m, vc, chunks[c])

    o_ref[...] = jnp.concatenate(chunks, axis=0)


def kernel(x, v, u):
    B, L = x.shape
    xT = x.astype(jnp.int32).T
    vT = v.T
    u_arr = jnp.full((1, 1), u, dtype=jnp.float32)

    out = pl.pallas_call(
        _rosa_block,
        grid=(B // _BBL,),
        in_specs=[
            pl.BlockSpec(memory_space=pltpu.SMEM),
            pl.BlockSpec((L, _BBL), lambda i: (0, i)),
            pl.BlockSpec((L, _BBL), lambda i: (0, i)),
        ],
        out_specs=pl.BlockSpec((L, _BBL), lambda i: (0, i)),
        out_shape=jax.ShapeDtypeStruct((L, B), jnp.float32),
        compiler_params=pltpu.CompilerParams(
            dimension_semantics=("parallel",)),
    )(u_arr, xT, vT)
    return out.T
